# baseline (device time: 64078 ns/iter reference)
import jax
import jax.numpy as jnp
from jax import lax
from jax.experimental import pallas as pl
from jax.experimental.pallas import tpu as pltpu

N_DEV = 16
SQ = 1024
HQ = 8
DH = 128
D = HQ * DH
BLK = 64
N_QB = SQ // BLK
N_PHASE = 4
SKV_SHARD = 1024
SCALE = 0.08838834764831843


def kernel(x, Wq, K_ext, V_ext, Wo):
    def body(x_ref, wq_ref, k_ref, v_ref, wo_ref, out_ref,
             send_q, recv_q, send_meta, recv_meta, out_comm,
             send_sems1, recv_sems1, send_sems2, recv_sems2,
             send_sems3, recv_sems3):
        my = lax.axis_index("i")

        xb = x_ref[0].astype(jnp.bfloat16)
        wqb = wq_ref[...].astype(jnp.bfloat16)
        q = jnp.dot(xb, wqb, preferred_element_type=jnp.float32)
        qb16 = q.astype(jnp.bfloat16)
        kb16 = k_ref[0].reshape(SKV_SHARD, D).astype(jnp.bfloat16)
        vb16 = v_ref[0].reshape(SKV_SHARD, D).astype(jnp.bfloat16)

        for p in range(N_PHASE):
            blocks = [p + N_PHASE * a for a in range(N_QB // N_PHASE)]
            qp = jnp.concatenate(
                [qb16[b * BLK:(b + 1) * BLK] for b in blocks], axis=0)
            kp = jnp.concatenate(
                [kb16[b * BLK:(b + 1) * BLK] for b in blocks], axis=0)
            vp = jnp.concatenate(
                [vb16[b * BLK:(b + 1) * BLK] for b in blocks], axis=0)
            ctx_h = []
            l_h = []
            for h in range(HQ):
                qh = qp[:, h * DH:(h + 1) * DH]
                kh = kp[:, h * DH:(h + 1) * DH]
                s = lax.dot_general(
                    qh, kh, (((1,), (1,)), ((), ())),
                    preferred_element_type=jnp.float32) * SCALE
                w = jnp.exp(s)
                l_h.append(jnp.sum(w, axis=1, keepdims=True))
                ctx_h.append(jnp.dot(
                    w.astype(jnp.bfloat16), vp[:, h * DH:(h + 1) * DH],
                    preferred_element_type=jnp.float32))
            l_p = jnp.concatenate(l_h, axis=1)

            for a in range(N_QB // N_PHASE):
                e = p + N_PHASE * a
                scales = []
                for h in range(HQ):
                    block = ctx_h[h][a * BLK:(a + 1) * BLK]
                    m = jnp.max(jnp.abs(block), axis=(0, 1),
                                keepdims=True) + 1e-20
                    inv = 127.0 / m
                    q8 = jnp.clip(jnp.floor(block * inv + 0.5),
                                  -127.0, 127.0).astype(jnp.int8)
                    send_q[e, :, h * DH:(h + 1) * DH] = q8
                    scales.append(m * (1.0 / 127.0))
                svec = jnp.concatenate(scales, axis=1)
                send_meta[e, :, :HQ] = (
                    l_p[a * BLK:(a + 1) * BLK].astype(jnp.bfloat16))
                send_meta[e, :, HQ:] = jnp.broadcast_to(
                    svec, (BLK, HQ)).astype(jnp.bfloat16)

        recv_q[pl.ds(my, 1)] = send_q[pl.ds(my, 1)]
        recv_meta[pl.ds(my, 1)] = send_meta[pl.ds(my, 1)]

        r1 = []
        for o in range(1, N_DEV):
            e = (my + o) % N_DEV
            rdma = pltpu.make_async_remote_copy(
                src_ref=send_q.at[e],
                dst_ref=recv_q.at[my],
                send_sem=send_sems1.at[o],
                recv_sem=recv_sems1.at[my],
                device_id=(e,),
                device_id_type=pl.DeviceIdType.MESH,
            )
            rdma.start()
            r1.append(rdma)
            rdma_m = pltpu.make_async_remote_copy(
                src_ref=send_meta.at[e],
                dst_ref=recv_meta.at[my],
                send_sem=send_sems3.at[o],
                recv_sem=recv_sems3.at[my],
                device_id=(e,),
                device_id_type=pl.DeviceIdType.MESH,
            )
            rdma_m.start()
            r1.append(rdma_m)
        for o in range(1, N_DEV):
            s = (my - o) % N_DEV
            rdma = pltpu.make_async_remote_copy(
                src_ref=send_q.at[s],
                dst_ref=recv_q.at[s],
                send_sem=send_sems1.at[o],
                recv_sem=recv_sems1.at[s],
                device_id=(s,),
                device_id_type=pl.DeviceIdType.MESH,
            )
            rdma.wait_recv()
            rdma_m = pltpu.make_async_remote_copy(
                src_ref=send_meta.at[s],
                dst_ref=recv_meta.at[s],
                send_sem=send_sems3.at[o],
                recv_sem=recv_sems3.at[s],
                device_id=(s,),
                device_id_type=pl.DeviceIdType.MESH,
            )
            rdma_m.wait_recv()

        ctx_sum = None
        l_sum = None
        for src in range(N_DEV):
            qf = recv_q[src].astype(jnp.float32)
            meta = recv_meta[src].astype(jnp.float32)
            sc = meta[:, HQ:]
            sc_full = jnp.concatenate(
                [jnp.broadcast_to(sc[:, h:h + 1], (BLK, DH))
                 for h in range(HQ)], axis=1)
            ctx_s = qf * sc_full
            l_s = meta[:, :HQ]
            ctx_sum = ctx_s if ctx_sum is None else ctx_sum + ctx_s
            l_sum = l_s if l_sum is None else l_sum + l_s
        attn = jnp.concatenate(
            [ctx_sum[:, h * DH:(h + 1) * DH] / l_sum[:, h:h + 1]
             for h in range(HQ)], axis=1)
        y = jnp.dot(attn.astype(jnp.bfloat16),
                    wo_ref[...].astype(jnp.bfloat16),
                    preferred_element_type=jnp.float32)
        out_ref[0, pl.ds(my * BLK, BLK), :] = y
        out_comm[pl.ds(my * BLK, BLK), :] = y.astype(jnp.bfloat16)

        r2 = []
        for o in range(1, N_DEV):
            e = (my + o) % N_DEV
            rdma = pltpu.make_async_remote_copy(
                src_ref=out_comm.at[pl.ds(my * BLK, BLK), :],
                dst_ref=out_comm.at[pl.ds(my * BLK, BLK), :],
                send_sem=send_sems2.at[o],
                recv_sem=recv_sems2.at[my],
                device_id=(e,),
                device_id_type=pl.DeviceIdType.MESH,
            )
            rdma.start()
            r2.append(rdma)
        for r in r1:
            r.wait_send()
        for o in range(1, N_DEV):
            s = (my - o) % N_DEV
            rdma = pltpu.make_async_remote_copy(
                src_ref=out_comm.at[pl.ds(s * BLK, BLK), :],
                dst_ref=out_comm.at[pl.ds(s * BLK, BLK), :],
                send_sem=send_sems2.at[o],
                recv_sem=recv_sems2.at[s],
                device_id=(s,),
                device_id_type=pl.DeviceIdType.MESH,
            )
            rdma.wait_recv()
            out_ref[0, pl.ds(s * BLK, BLK), :] = (
                out_comm[pl.ds(s * BLK, BLK), :].astype(jnp.float32))
        for r in r2:
            r.wait_send()

    return pl.pallas_call(
        body,
        out_shape=jax.ShapeDtypeStruct((1, SQ, SQ), jnp.float32),
        in_specs=[pl.BlockSpec(memory_space=pltpu.VMEM)] * 5,
        out_specs=pl.BlockSpec(memory_space=pltpu.VMEM),
        scratch_shapes=[
            pltpu.VMEM((N_DEV, BLK, D), jnp.int8),
            pltpu.VMEM((N_DEV, BLK, D), jnp.int8),
            pltpu.VMEM((N_DEV, BLK, 2 * HQ), jnp.bfloat16),
            pltpu.VMEM((N_DEV, BLK, 2 * HQ), jnp.bfloat16),
            pltpu.VMEM((SQ, D), jnp.bfloat16),
            pltpu.SemaphoreType.DMA((N_DEV,)),
            pltpu.SemaphoreType.DMA((N_DEV,)),
            pltpu.SemaphoreType.DMA((N_DEV,)),
            pltpu.SemaphoreType.DMA((N_DEV,)),
            pltpu.SemaphoreType.DMA((N_DEV,)),
            pltpu.SemaphoreType.DMA((N_DEV,)),
        ],
    )(x, Wq, K_ext, V_ext, Wo)


# device time: 57106 ns/iter; 1.1221x vs baseline; 1.1221x over previous
import jax
import jax.numpy as jnp
from jax import lax
from jax.experimental import pallas as pl
from jax.experimental.pallas import tpu as pltpu

N_DEV = 16
SQ = 1024
HQ = 8
DH = 128
D = HQ * DH
BLK = 64
N_QB = SQ // BLK
N_PHASE = 4
SKV_SHARD = 1024
SCALE = 0.08838834764831843


def kernel(x, Wq, K_ext, V_ext, Wo):
    def body(x_ref, wq_ref, k_ref, v_ref, wo_ref, out_ref,
             send_q, recv_q, send_meta, recv_meta, out_q, y_meta,
             send_sems1, recv_sems1, send_sems2, recv_sems2,
             send_sems3, recv_sems3, send_sems4, recv_sems4):
        my = lax.axis_index("i")

        xb = x_ref[0].astype(jnp.bfloat16)
        wqb = wq_ref[...].astype(jnp.bfloat16)
        q = jnp.dot(xb, wqb, preferred_element_type=jnp.float32)
        qb16 = q.astype(jnp.bfloat16)
        kb16 = k_ref[0].reshape(SKV_SHARD, D).astype(jnp.bfloat16)
        vb16 = v_ref[0].reshape(SKV_SHARD, D).astype(jnp.bfloat16)

        for p in range(N_PHASE):
            blocks = [p + N_PHASE * a for a in range(N_QB // N_PHASE)]
            qp = jnp.concatenate(
                [qb16[b * BLK:(b + 1) * BLK] for b in blocks], axis=0)
            kp = jnp.concatenate(
                [kb16[b * BLK:(b + 1) * BLK] for b in blocks], axis=0)
            vp = jnp.concatenate(
                [vb16[b * BLK:(b + 1) * BLK] for b in blocks], axis=0)
            ctx_h = []
            l_h = []
            for h in range(HQ):
                qh = qp[:, h * DH:(h + 1) * DH]
                kh = kp[:, h * DH:(h + 1) * DH]
                s = lax.dot_general(
                    qh, kh, (((1,), (1,)), ((), ())),
                    preferred_element_type=jnp.float32) * SCALE
                w = jnp.exp(s)
                l_h.append(jnp.sum(w, axis=1, keepdims=True))
                ctx_h.append(jnp.dot(
                    w.astype(jnp.bfloat16), vp[:, h * DH:(h + 1) * DH],
                    preferred_element_type=jnp.float32))
            l_p = jnp.concatenate(l_h, axis=1)

            for a in range(N_QB // N_PHASE):
                e = p + N_PHASE * a
                scales = []
                for h in range(HQ):
                    block = ctx_h[h][a * BLK:(a + 1) * BLK]
                    m = jnp.max(jnp.abs(block), axis=(0, 1),
                                keepdims=True) + 1e-20
                    inv = 127.0 / m
                    q8 = jnp.clip(jnp.floor(block * inv + 0.5),
                                  -127.0, 127.0).astype(jnp.int8)
                    send_q[e, :, h * DH:(h + 1) * DH] = q8
                    scales.append(m * (1.0 / 127.0))
                svec = jnp.concatenate(scales, axis=1)
                send_meta[e, :, :HQ] = (
                    l_p[a * BLK:(a + 1) * BLK].astype(jnp.bfloat16))
                send_meta[e, :, HQ:] = jnp.broadcast_to(
                    svec, (BLK, HQ)).astype(jnp.bfloat16)

        recv_q[pl.ds(my, 1)] = send_q[pl.ds(my, 1)]
        recv_meta[pl.ds(my, 1)] = send_meta[pl.ds(my, 1)]

        r1 = []
        for o in range(1, N_DEV):
            e = (my + o) % N_DEV
            rdma = pltpu.make_async_remote_copy(
                src_ref=send_q.at[e],
                dst_ref=recv_q.at[my],
                send_sem=send_sems1.at[o],
                recv_sem=recv_sems1.at[my],
                device_id=(e,),
                device_id_type=pl.DeviceIdType.MESH,
            )
            rdma.start()
            r1.append(rdma)
            rdma_m = pltpu.make_async_remote_copy(
                src_ref=send_meta.at[e],
                dst_ref=recv_meta.at[my],
                send_sem=send_sems3.at[o],
                recv_sem=recv_sems3.at[my],
                device_id=(e,),
                device_id_type=pl.DeviceIdType.MESH,
            )
            rdma_m.start()
            r1.append(rdma_m)
        for o in range(1, N_DEV):
            s = (my - o) % N_DEV
            rdma = pltpu.make_async_remote_copy(
                src_ref=send_q.at[s],
                dst_ref=recv_q.at[s],
                send_sem=send_sems1.at[o],
                recv_sem=recv_sems1.at[s],
                device_id=(s,),
                device_id_type=pl.DeviceIdType.MESH,
            )
            rdma.wait_recv()
            rdma_m = pltpu.make_async_remote_copy(
                src_ref=send_meta.at[s],
                dst_ref=recv_meta.at[s],
                send_sem=send_sems3.at[o],
                recv_sem=recv_sems3.at[s],
                device_id=(s,),
                device_id_type=pl.DeviceIdType.MESH,
            )
            rdma_m.wait_recv()

        ctx_sum = None
        l_sum = None
        for src in range(N_DEV):
            qf = recv_q[src].astype(jnp.float32)
            meta = recv_meta[src].astype(jnp.float32)
            sc = meta[:, HQ:]
            sc_full = jnp.concatenate(
                [jnp.broadcast_to(sc[:, h:h + 1], (BLK, DH))
                 for h in range(HQ)], axis=1)
            ctx_s = qf * sc_full
            l_s = meta[:, :HQ]
            ctx_sum = ctx_s if ctx_sum is None else ctx_sum + ctx_s
            l_sum = l_s if l_sum is None else l_sum + l_s
        attn = jnp.concatenate(
            [ctx_sum[:, h * DH:(h + 1) * DH] / l_sum[:, h:h + 1]
             for h in range(HQ)], axis=1)
        y = jnp.dot(attn.astype(jnp.bfloat16),
                    wo_ref[...].astype(jnp.bfloat16),
                    preferred_element_type=jnp.float32)
        out_ref[0, pl.ds(my * BLK, BLK), :] = y

        y_scales = []
        for g in range(HQ):
            blk = y[:, g * DH:(g + 1) * DH]
            m = jnp.max(jnp.abs(blk), axis=(0, 1), keepdims=True) + 1e-20
            inv = 127.0 / m
            q8 = jnp.clip(jnp.floor(blk * inv + 0.5),
                          -127.0, 127.0).astype(jnp.int8)
            out_q[pl.ds(my * BLK, BLK), g * DH:(g + 1) * DH] = q8
            y_scales.append(m * (1.0 / 127.0))
        ysvec = jnp.concatenate(y_scales, axis=1)
        y_meta[pl.ds(my, 1)] = jnp.broadcast_to(
            ysvec, (8, HQ)).astype(jnp.bfloat16).reshape(1, 8, HQ)

        r2 = []
        for o in range(1, N_DEV):
            e = (my + o) % N_DEV
            rdma = pltpu.make_async_remote_copy(
                src_ref=out_q.at[pl.ds(my * BLK, BLK), :],
                dst_ref=out_q.at[pl.ds(my * BLK, BLK), :],
                send_sem=send_sems2.at[o],
                recv_sem=recv_sems2.at[my],
                device_id=(e,),
                device_id_type=pl.DeviceIdType.MESH,
            )
            rdma.start()
            r2.append(rdma)
            rdma_m = pltpu.make_async_remote_copy(
                src_ref=y_meta.at[my],
                dst_ref=y_meta.at[my],
                send_sem=send_sems4.at[o],
                recv_sem=recv_sems4.at[my],
                device_id=(e,),
                device_id_type=pl.DeviceIdType.MESH,
            )
            rdma_m.start()
            r2.append(rdma_m)
        for r in r1:
            r.wait_send()
        for o in range(1, N_DEV):
            s = (my - o) % N_DEV
            rdma = pltpu.make_async_remote_copy(
                src_ref=out_q.at[pl.ds(s * BLK, BLK), :],
                dst_ref=out_q.at[pl.ds(s * BLK, BLK), :],
                send_sem=send_sems2.at[o],
                recv_sem=recv_sems2.at[s],
                device_id=(s,),
                device_id_type=pl.DeviceIdType.MESH,
            )
            rdma.wait_recv()
            rdma_m = pltpu.make_async_remote_copy(
                src_ref=y_meta.at[s],
                dst_ref=y_meta.at[s],
                send_sem=send_sems4.at[o],
                recv_sem=recv_sems4.at[s],
                device_id=(s,),
                device_id_type=pl.DeviceIdType.MESH,
            )
            rdma_m.wait_recv()
            qf = out_q[pl.ds(s * BLK, BLK), :].astype(jnp.float32)
            sc = y_meta[s].astype(jnp.float32)
            sc_full = jnp.concatenate(
                [jnp.broadcast_to(sc[0:1, g:g + 1], (BLK, DH))
                 for g in range(HQ)], axis=1)
            out_ref[0, pl.ds(s * BLK, BLK), :] = qf * sc_full
        for r in r2:
            r.wait_send()

    return pl.pallas_call(
        body,
        out_shape=jax.ShapeDtypeStruct((1, SQ, SQ), jnp.float32),
        in_specs=[pl.BlockSpec(memory_space=pltpu.VMEM)] * 5,
        out_specs=pl.BlockSpec(memory_space=pltpu.VMEM),
        scratch_shapes=[
            pltpu.VMEM((N_DEV, BLK, D), jnp.int8),
            pltpu.VMEM((N_DEV, BLK, D), jnp.int8),
            pltpu.VMEM((N_DEV, BLK, 2 * HQ), jnp.bfloat16),
            pltpu.VMEM((N_DEV, BLK, 2 * HQ), jnp.bfloat16),
            pltpu.VMEM((SQ, D), jnp.int8),
            pltpu.VMEM((N_DEV, 8, HQ), jnp.bfloat16),
            pltpu.SemaphoreType.DMA((N_DEV,)),
            pltpu.SemaphoreType.DMA((N_DEV,)),
            pltpu.SemaphoreType.DMA((N_DEV,)),
            pltpu.SemaphoreType.DMA((N_DEV,)),
            pltpu.SemaphoreType.DMA((N_DEV,)),
            pltpu.SemaphoreType.DMA((N_DEV,)),
            pltpu.SemaphoreType.DMA((N_DEV,)),
            pltpu.SemaphoreType.DMA((N_DEV,)),
        ],
    )(x, Wq, K_ext, V_ext, Wo)
